# baseline (speedup 1.0000x reference)
import functools

import jax
import jax.numpy as jnp
from jax import lax
from jax.experimental import pallas as pl
from jax.experimental.pallas import tpu as pltpu
from jax.experimental.pallas import tpu_sc as plsc

_LANES = 16
_NC = 2
_NS = 16
_NW = _NC * _NS
_RING = 15


def kernel(user_id, item_id, users_table, items_table):
    batch = user_id.shape[0]
    vocab, latent = users_table.shape
    bpw = batch // _NW

    uid = user_id.astype(jnp.int32)
    iid = item_id.astype(jnp.int32)
    ut_t = users_table.T
    it_t = items_table.T

    @functools.partial(
        pl.kernel,
        out_type=jax.ShapeDtypeStruct((batch,), jnp.float32),
        mesh=plsc.VectorSubcoreMesh(core_axis_name="c", subcore_axis_name="s"),
        compiler_params=pltpu.CompilerParams(
            needs_layout_passes=False, use_tc_tiling_on_sc=True),
        scratch_types=[
            pltpu.VMEM((bpw,), jnp.int32),
            pltpu.VMEM((bpw,), jnp.int32),
            pltpu.SMEM((bpw,), jnp.float32),
            pltpu.VMEM((_RING, latent, 128), jnp.float32),
            pltpu.VMEM((_RING, latent, 128), jnp.float32),
            pltpu.VMEM((bpw,), jnp.float32),
            pltpu.SemaphoreType.DMA((_RING,)),
            pltpu.SemaphoreType.DMA((_RING,)),
        ],
    )
    def mf(uid_hbm, iid_hbm, ut_hbm, it_hbm, out_hbm,
           uids, iids, outs, uwin, vwin, outv, usem, vsem):
        wid = lax.axis_index("s") * _NC + lax.axis_index("c")
        base = wid * bpw
        pltpu.sync_copy(uid_hbm.at[pl.ds(base, bpw)], uids)
        pltpu.sync_copy(iid_hbm.at[pl.ds(base, bpw)], iids)

        lane = lax.iota(jnp.int32, _LANES)
        lane_hi = lane + _LANES
        zero = jnp.zeros((_LANES,), jnp.int32)

        def scalar_at(vec, mask):
            return jnp.sum(jnp.where(mask, vec, zero))

        def compute_scalars(i0, nb):
            us, vs = [], []
            for b in range(nb):
                e = jnp.minimum(i0 + b, bpw - 1)
                vbase = (e // _LANES) * _LANES
                uvec = uids[pl.ds(vbase, _LANES)]
                vvec = iids[pl.ds(vbase, _LANES)]
                mask = lane == (e % _LANES)
                us.append(scalar_at(uvec, mask))
                vs.append(scalar_at(vvec, mask))
            return tuple(us) + tuple(vs)

        def process_batch(i0, nb, scal, prefetch_next):
            us, vs = scal[:nb], scal[nb:]
            copies = []
            lanes_u = []
            lanes_v = []
            for b in range(nb):
                uoff = pl.multiple_of(
                    lax.shift_left(lax.shift_right_logical(us[b], 7), 7), 128)
                voff = pl.multiple_of(
                    lax.shift_left(lax.shift_right_logical(vs[b], 7), 7), 128)
                lanes_u.append(
                    jnp.full((_LANES,), jnp.bitwise_and(us[b], 127)))
                lanes_v.append(
                    jnp.full((_LANES,), jnp.bitwise_and(vs[b], 127)))
                copies.append((
                    pltpu.async_copy(
                        ut_hbm.at[:, pl.ds(uoff, 128)], uwin.at[b],
                        usem.at[b]),
                    pltpu.async_copy(
                        it_hbm.at[:, pl.ds(voff, 128)], vwin.at[b],
                        vsem.at[b]),
                ))

            nxt = compute_scalars(i0 + nb, nb) if prefetch_next else ()

            for b in range(nb):
                cu, cv = copies[b]
                cu.wait()
                cv.wait()
                bb = jnp.full((_LANES,), b, jnp.int32)
                ulo = plsc.load_gather(uwin, [bb, lane, lanes_u[b]])
                uhi = plsc.load_gather(uwin, [bb, lane_hi, lanes_u[b]])
                vlo = plsc.load_gather(vwin, [bb, lane, lanes_v[b]])
                vhi = plsc.load_gather(vwin, [bb, lane_hi, lanes_v[b]])
                prod = ulo * vlo + uhi * vhi
                outs[i0 + b] = jnp.sum(prod)
            return nxt

        def body(g, scal):
            return process_batch(g * _RING, _RING, scal, True)

        n_full = bpw // _RING
        lax.fori_loop(0, n_full, body, compute_scalars(0, _RING))
        if bpw % _RING:
            tail = bpw % _RING
            process_batch(n_full * _RING, tail,
                          compute_scalars(n_full * _RING, tail), False)

        def pack(g, carry):
            vals = jnp.zeros((_LANES,), jnp.float32)
            for j in range(_LANES):
                s = outs[g * _LANES + j]
                vals = jnp.where(lane == j, jnp.full((_LANES,), s), vals)
            outv[pl.ds(g * _LANES, _LANES)] = vals
            return carry

        lax.fori_loop(0, bpw // _LANES, pack, 0)
        pltpu.sync_copy(outv, out_hbm.at[pl.ds(base, bpw)])

    return mf(uid, iid, ut_t, it_t)
